# baseline (speedup 1.0000x reference)
import functools

import jax
import jax.numpy as jnp
from jax import lax
from jax.experimental import pallas as pl
from jax.experimental.pallas import tpu as pltpu
from jax.experimental.pallas import tpu_sc as plsc

N = 10000
D = 128
NCORES = 2
NSUB = 16
NW = NCORES * NSUB
BLK = 128
NBLK = 80
NPAD = N + 112
RPS = NPAD // NSUB
LASTR = N - (NSUB - 1) * RPS
RB = 1000


@functools.partial(
    pl.kernel,
    out_type=jax.ShapeDtypeStruct((NCORES, N, D), jnp.float32),
    mesh=plsc.VectorSubcoreMesh(core_axis_name="c", subcore_axis_name="s"),
    scratch_types=[
        pltpu.VMEM((1, BLK), jnp.int32),
        pltpu.VMEM((1, BLK), jnp.int32),
        pltpu.VMEM((BLK, D), jnp.float32),
        pltpu.VMEM_SHARED((NPAD, D), jnp.float32),
        pltpu.SemaphoreType.DMA,
    ],
)
def _agg(y_hbm, src_hbm, dst_hbm, zrows_hbm, out_hbm, idx_s, idx_d, rows, acc, sem):
    c = lax.axis_index("c")
    s = lax.axis_index("s")
    wid = s * NCORES + c
    base = wid * (NBLK * BLK)

    @pl.when(jnp.logical_and(c == 0, s < NSUB - 1))
    def _():
        pltpu.sync_copy(y_hbm.at[pl.ds(s * RPS, RPS)], acc.at[pl.ds(s * RPS, RPS)])

    @pl.when(jnp.logical_and(c == 0, s == NSUB - 1))
    def _():
        pltpu.sync_copy(y_hbm.at[pl.ds(s * RPS, LASTR)], acc.at[pl.ds(s * RPS, LASTR)])

    @pl.when(jnp.logical_and(c != 0, s < NSUB - 1))
    def _():
        pltpu.sync_copy(zrows_hbm, acc.at[pl.ds(s * RPS, RPS)])

    @pl.when(jnp.logical_and(c != 0, s == NSUB - 1))
    def _():
        pltpu.sync_copy(zrows_hbm.at[pl.ds(0, LASTR)], acc.at[pl.ds(s * RPS, LASTR)])

    plsc.subcore_barrier()

    def eblk(i, carry):
        off = base + i * BLK
        pltpu.sync_copy(src_hbm.at[pl.ds(off, BLK)], idx_s.at[0])
        pltpu.sync_copy(dst_hbm.at[pl.ds(off, BLK)], idx_d.at[0])
        pltpu.async_copy(y_hbm.at[idx_s.at[0]], rows, sem).wait()
        pltpu.sync_copy(rows, acc.at[idx_d.at[0]], add=True)
        return carry

    lax.fori_loop(0, NBLK, eblk, 0)
    plsc.subcore_barrier()

    @pl.when(s < NSUB - 1)
    def _():
        pltpu.sync_copy(acc.at[pl.ds(s * RPS, RPS)], out_hbm.at[c, pl.ds(s * RPS, RPS)])

    @pl.when(s == NSUB - 1)
    def _():
        pltpu.sync_copy(acc.at[pl.ds(s * RPS, LASTR)], out_hbm.at[c, pl.ds(s * RPS, LASTR)])


@functools.partial(
    pl.kernel,
    out_type=jax.ShapeDtypeStruct((NCORES * NPAD,), jnp.float32),
    mesh=plsc.VectorSubcoreMesh(core_axis_name="c", subcore_axis_name="s"),
    scratch_types=[
        pltpu.VMEM((2, BLK), jnp.int32),
        pltpu.VMEM((1, BLK), jnp.float32),
        pltpu.VMEM((NPAD // NSUB,), jnp.float32),
        pltpu.VMEM_SHARED((NPAD,), jnp.float32),
        pltpu.SemaphoreType.DMA,
        pltpu.SemaphoreType.DMA,
    ],
)
def _deg(dst_hbm, znpad_hbm, ones_hbm, out_hbm, idxb, onesb, bounce, acc1, d0, d1):
    c = lax.axis_index("c")
    s = lax.axis_index("s")
    wid = s * NCORES + c
    base = wid * (NBLK * BLK)
    eps = NPAD // NSUB
    pltpu.sync_copy(ones_hbm, onesb)
    pltpu.sync_copy(znpad_hbm.at[pl.ds(s * eps, eps)], bounce)
    pltpu.sync_copy(bounce, acc1.at[pl.ds(s * eps, eps)])
    plsc.subcore_barrier()

    pltpu.async_copy(dst_hbm.at[pl.ds(base, BLK)], idxb.at[0], d0)
    pltpu.async_copy(dst_hbm.at[pl.ds(base + BLK, BLK)], idxb.at[1], d1)

    def pair(gi, carry):
        i0 = 2 * gi
        i1 = i0 + 1
        pltpu.make_async_copy(dst_hbm.at[pl.ds(base, BLK)], idxb.at[0], d0).wait()
        pltpu.sync_copy(onesb.at[0], acc1.at[idxb.at[0]], add=True)

        @pl.when(i0 + 2 < NBLK)
        def _():
            pltpu.async_copy(dst_hbm.at[pl.ds(base + (i0 + 2) * BLK, BLK)], idxb.at[0], d0)

        pltpu.make_async_copy(dst_hbm.at[pl.ds(base, BLK)], idxb.at[1], d1).wait()
        pltpu.sync_copy(onesb.at[0], acc1.at[idxb.at[1]], add=True)

        @pl.when(i1 + 2 < NBLK)
        def _():
            pltpu.async_copy(dst_hbm.at[pl.ds(base + (i1 + 2) * BLK, BLK)], idxb.at[1], d1)

        return carry

    lax.fori_loop(0, NBLK // 2, pair, 0)
    plsc.subcore_barrier()
    pltpu.sync_copy(acc1.at[pl.ds(s * eps, eps)], bounce)
    pltpu.sync_copy(bounce, out_hbm.at[pl.ds(c * NPAD + s * eps, eps)])


def _dinv_of(degp_ref):
    return lax.rsqrt(degp_ref[0] + degp_ref[1] + 1.0)


def _tc_first_body(x_ref, degp_ref, w_ref, y_ref):
    dinv = _dinv_of(degp_ref)
    y_ref[...] = jnp.dot(x_ref[...], w_ref[...],
                         preferred_element_type=jnp.float32) * dinv


_tc_first = pl.pallas_call(
    _tc_first_body,
    grid=(N // RB,),
    in_specs=[
        pl.BlockSpec((RB, D), lambda i: (i, 0)),
        pl.BlockSpec((NCORES, RB, 1), lambda i: (0, i, 0)),
        pl.BlockSpec((D, D), lambda i: (0, 0)),
    ],
    out_specs=pl.BlockSpec((RB, D), lambda i: (i, 0)),
    out_shape=jax.ShapeDtypeStruct((N, D), jnp.float32),
)


def _tc_mid_body(s2_ref, degp_ref, b_ref, w_ref, y_ref):
    dinv = _dinv_of(degp_ref)
    sagg = s2_ref[0] + s2_ref[1]
    h = jnp.maximum(sagg * dinv + b_ref[0][None, :], 0.0)
    y_ref[...] = jnp.dot(h, w_ref[...],
                         preferred_element_type=jnp.float32) * dinv


_tc_mid = pl.pallas_call(
    _tc_mid_body,
    grid=(N // RB,),
    in_specs=[
        pl.BlockSpec((NCORES, RB, D), lambda i: (0, i, 0)),
        pl.BlockSpec((NCORES, RB, 1), lambda i: (0, i, 0)),
        pl.BlockSpec((1, D), lambda i: (0, 0)),
        pl.BlockSpec((D, D), lambda i: (0, 0)),
    ],
    out_specs=pl.BlockSpec((RB, D), lambda i: (i, 0)),
    out_shape=jax.ShapeDtypeStruct((N, D), jnp.float32),
)


def _make_tc_final(ncls):
    def body(s2_ref, degp_ref, b_ref, out_ref):
        dinv = _dinv_of(degp_ref)
        logits = (s2_ref[0] + s2_ref[1]) * dinv + b_ref[0][None, :]
        col = lax.broadcasted_iota(jnp.int32, (RB, D), 1)
        masked = jnp.where(col < ncls, logits, -1e30)
        m = jnp.max(masked, axis=-1, keepdims=True)
        lse = m + jnp.log(jnp.sum(jnp.exp(masked - m), axis=-1, keepdims=True))
        out_ref[...] = logits - lse

    return pl.pallas_call(
        body,
        grid=(N // RB,),
        in_specs=[
            pl.BlockSpec((NCORES, RB, D), lambda i: (0, i, 0)),
            pl.BlockSpec((NCORES, RB, 1), lambda i: (0, i, 0)),
            pl.BlockSpec((1, D), lambda i: (0, 0)),
        ],
        out_specs=pl.BlockSpec((RB, D), lambda i: (i, 0)),
        out_shape=jax.ShapeDtypeStruct((N, D), jnp.float32),
    )


def kernel(x, edge_index, W1, b1, W2, b2, W3, b3, W4, b4, W5, b5, W6, b6):
    ncls = W6.shape[1]
    w6p = jnp.pad(W6, ((0, 0), (0, D - ncls)))
    b6p = jnp.pad(b6, (0, D - ncls))
    src = edge_index[0]
    dst = edge_index[1]
    e = src.shape[0]
    pad = NW * NBLK * BLK - e
    src_p = jnp.concatenate([src, jnp.zeros((pad,), src.dtype)])
    dst_p = jnp.concatenate([dst, jnp.full((pad,), N, dst.dtype)])
    zrows = jnp.zeros((RPS, D), jnp.float32)
    znpad = jnp.zeros((NPAD,), jnp.float32)

    degp = _deg(dst_p, znpad, jnp.ones((1, BLK), jnp.float32)).reshape(NCORES, NPAD, 1)
    y = _tc_first(x, degp, W1)
    for bk, wn in ((b1, W2), (b2, W3), (b3, W4), (b4, W5), (b5, w6p)):
        s2 = _agg(y, src_p, dst_p, zrows)
        y = _tc_mid(s2, degp, bk.reshape(1, D), wn)
    s2 = _agg(y, src_p, dst_p, zrows)
    out = _make_tc_final(ncls)(s2, degp, b6p.reshape(1, D))
    return out[:, :ncls]


# speedup vs baseline: 2.2509x; 2.2509x over previous
import functools

import jax
import jax.numpy as jnp
from jax import lax
from jax.experimental import pallas as pl
from jax.experimental.pallas import tpu as pltpu
from jax.experimental.pallas import tpu_sc as plsc

N = 10000
D = 128
NCORES = 2
NSUB = 16
NW = NCORES * NSUB
BLK = 128
NBLK = 80
NPAD = N + 112
RPS = NPAD // NSUB
LASTR = N - (NSUB - 1) * RPS
RB = 1000


@functools.partial(
    pl.kernel,
    out_type=jax.ShapeDtypeStruct((NCORES, N, D), jnp.float32),
    mesh=plsc.VectorSubcoreMesh(core_axis_name="c", subcore_axis_name="s"),
    scratch_types=[
        pltpu.VMEM((1, BLK), jnp.int32),
        pltpu.VMEM((1, BLK), jnp.int32),
        pltpu.VMEM((BLK, D), jnp.float32),
        pltpu.VMEM_SHARED((NPAD, D), jnp.float32),
        pltpu.SemaphoreType.DMA,
    ],
)
def _agg(y_hbm, src_hbm, dst_hbm, zrows_hbm, out_hbm, idx_s, idx_d, rows, acc, sem):
    c = lax.axis_index("c")
    s = lax.axis_index("s")
    wid = s * NCORES + c
    base = wid * (NBLK * BLK)

    @pl.when(jnp.logical_and(c == 0, s < NSUB - 1))
    def _():
        pltpu.sync_copy(y_hbm.at[pl.ds(s * RPS, RPS)], acc.at[pl.ds(s * RPS, RPS)])

    @pl.when(jnp.logical_and(c == 0, s == NSUB - 1))
    def _():
        pltpu.sync_copy(y_hbm.at[pl.ds(s * RPS, LASTR)], acc.at[pl.ds(s * RPS, LASTR)])

    @pl.when(jnp.logical_and(c != 0, s < NSUB - 1))
    def _():
        pltpu.sync_copy(zrows_hbm, acc.at[pl.ds(s * RPS, RPS)])

    @pl.when(jnp.logical_and(c != 0, s == NSUB - 1))
    def _():
        pltpu.sync_copy(zrows_hbm.at[pl.ds(0, LASTR)], acc.at[pl.ds(s * RPS, LASTR)])

    plsc.subcore_barrier()

    def eblk(i, carry):
        off = base + i * BLK
        pltpu.sync_copy(src_hbm.at[pl.ds(off, BLK)], idx_s.at[0])
        pltpu.sync_copy(dst_hbm.at[pl.ds(off, BLK)], idx_d.at[0])
        pltpu.async_copy(y_hbm.at[idx_s.at[0]], rows, sem).wait()
        pltpu.sync_copy(rows, acc.at[idx_d.at[0]], add=True)
        return carry

    lax.fori_loop(0, NBLK, eblk, 0)
    plsc.subcore_barrier()

    @pl.when(s < NSUB - 1)
    def _():
        pltpu.sync_copy(acc.at[pl.ds(s * RPS, RPS)], out_hbm.at[c, pl.ds(s * RPS, RPS)])

    @pl.when(s == NSUB - 1)
    def _():
        pltpu.sync_copy(acc.at[pl.ds(s * RPS, LASTR)], out_hbm.at[c, pl.ds(s * RPS, LASTR)])


@functools.partial(
    pl.kernel,
    out_type=jax.ShapeDtypeStruct((NCORES * NPAD,), jnp.float32),
    mesh=plsc.VectorSubcoreMesh(core_axis_name="c", subcore_axis_name="s"),
    scratch_types=[
        pltpu.VMEM((2, BLK), jnp.int32),
        pltpu.VMEM((1, BLK), jnp.float32),
        pltpu.VMEM((NPAD // NSUB,), jnp.float32),
        pltpu.VMEM_SHARED((NPAD,), jnp.float32),
        pltpu.SemaphoreType.DMA,
        pltpu.SemaphoreType.DMA,
    ],
)
def _deg(dst_hbm, znpad_hbm, ones_hbm, out_hbm, idxb, onesb, bounce, acc1, d0, d1):
    c = lax.axis_index("c")
    s = lax.axis_index("s")
    wid = s * NCORES + c
    base = wid * (NBLK * BLK)
    eps = NPAD // NSUB
    pltpu.sync_copy(ones_hbm, onesb)
    pltpu.sync_copy(znpad_hbm.at[pl.ds(s * eps, eps)], bounce)
    pltpu.sync_copy(bounce, acc1.at[pl.ds(s * eps, eps)])
    plsc.subcore_barrier()

    pltpu.async_copy(dst_hbm.at[pl.ds(base, BLK)], idxb.at[0], d0)
    pltpu.async_copy(dst_hbm.at[pl.ds(base + BLK, BLK)], idxb.at[1], d1)

    def pair(gi, carry):
        i0 = 2 * gi
        i1 = i0 + 1
        pltpu.make_async_copy(dst_hbm.at[pl.ds(base, BLK)], idxb.at[0], d0).wait()
        pltpu.sync_copy(onesb.at[0], acc1.at[idxb.at[0]], add=True)

        @pl.when(i0 + 2 < NBLK)
        def _():
            pltpu.async_copy(dst_hbm.at[pl.ds(base + (i0 + 2) * BLK, BLK)], idxb.at[0], d0)

        pltpu.make_async_copy(dst_hbm.at[pl.ds(base, BLK)], idxb.at[1], d1).wait()
        pltpu.sync_copy(onesb.at[0], acc1.at[idxb.at[1]], add=True)

        @pl.when(i1 + 2 < NBLK)
        def _():
            pltpu.async_copy(dst_hbm.at[pl.ds(base + (i1 + 2) * BLK, BLK)], idxb.at[1], d1)

        return carry

    lax.fori_loop(0, NBLK // 2, pair, 0)
    plsc.subcore_barrier()
    pltpu.sync_copy(acc1.at[pl.ds(s * eps, eps)], bounce)
    pltpu.sync_copy(bounce, out_hbm.at[pl.ds(c * NPAD + s * eps, eps)])


def _dinv_of(degp_ref):
    return lax.rsqrt(degp_ref[0] + degp_ref[1] + 1.0)


def _tc_first_body(x_ref, degp_ref, w_ref, y_ref):
    dinv = _dinv_of(degp_ref)
    y_ref[...] = jnp.dot(x_ref[...], w_ref[...],
                         preferred_element_type=jnp.float32) * dinv


_tc_first = pl.pallas_call(
    _tc_first_body,
    grid=(N // RB,),
    in_specs=[
        pl.BlockSpec((RB, D), lambda i: (i, 0)),
        pl.BlockSpec((NCORES, RB, 1), lambda i: (0, i, 0)),
        pl.BlockSpec((D, D), lambda i: (0, 0)),
    ],
    out_specs=pl.BlockSpec((RB, D), lambda i: (i, 0)),
    out_shape=jax.ShapeDtypeStruct((N, D), jnp.float32),
)


def _tc_mid_body(s2_ref, degp_ref, b_ref, w_ref, y_ref):
    dinv = _dinv_of(degp_ref)
    sagg = s2_ref[0] + s2_ref[1]
    h = jnp.maximum(sagg * dinv + b_ref[0][None, :], 0.0)
    y_ref[...] = jnp.dot(h, w_ref[...],
                         preferred_element_type=jnp.float32) * dinv


_tc_mid = pl.pallas_call(
    _tc_mid_body,
    grid=(N // RB,),
    in_specs=[
        pl.BlockSpec((NCORES, RB, D), lambda i: (0, i, 0)),
        pl.BlockSpec((NCORES, RB, 1), lambda i: (0, i, 0)),
        pl.BlockSpec((1, D), lambda i: (0, 0)),
        pl.BlockSpec((D, D), lambda i: (0, 0)),
    ],
    out_specs=pl.BlockSpec((RB, D), lambda i: (i, 0)),
    out_shape=jax.ShapeDtypeStruct((N, D), jnp.float32),
)


def _make_tc_final(ncls):
    def body(s2_ref, degp_ref, b_ref, out_ref):
        dinv = _dinv_of(degp_ref)
        logits = (s2_ref[0] + s2_ref[1]) * dinv + b_ref[0][None, :]
        col = lax.broadcasted_iota(jnp.int32, (RB, D), 1)
        masked = jnp.where(col < ncls, logits, -1e30)
        m = jnp.max(masked, axis=-1, keepdims=True)
        lse = m + jnp.log(jnp.sum(jnp.exp(masked - m), axis=-1, keepdims=True))
        out_ref[...] = logits - lse

    return pl.pallas_call(
        body,
        grid=(N // RB,),
        in_specs=[
            pl.BlockSpec((NCORES, RB, D), lambda i: (0, i, 0)),
            pl.BlockSpec((NCORES, RB, 1), lambda i: (0, i, 0)),
            pl.BlockSpec((1, D), lambda i: (0, 0)),
        ],
        out_specs=pl.BlockSpec((RB, D), lambda i: (i, 0)),
        out_shape=jax.ShapeDtypeStruct((N, D), jnp.float32),
    )


def kernel(x, edge_index, W1, b1, W2, b2, W3, b3, W4, b4, W5, b5, W6, b6):
    ncls = W6.shape[1]
    w6p = jnp.pad(W6, ((0, 0), (0, D - ncls)))
    b6p = jnp.pad(b6, (0, D - ncls))
    src = edge_index[0]
    dst = edge_index[1]
    e = src.shape[0]
    pad = NW * NBLK * BLK - e
    it = jnp.arange(pad, dtype=src.dtype)
    src_p = jnp.concatenate([src, it % BLK])
    dst_p = jnp.concatenate([dst, N + it % (NPAD - N)])
    zrows = jnp.zeros((RPS, D), jnp.float32)
    znpad = jnp.zeros((NPAD,), jnp.float32)

    degp = _deg(dst_p, znpad, jnp.ones((1, BLK), jnp.float32)).reshape(NCORES, NPAD, 1)
    y = _tc_first(x, degp, W1)
    for bk, wn in ((b1, W2), (b2, W3), (b3, W4), (b4, W5), (b5, w6p)):
        s2 = _agg(y, src_p, dst_p, zrows)
        y = _tc_mid(s2, degp, bk.reshape(1, D), wn)
    s2 = _agg(y, src_p, dst_p, zrows)
    out = _make_tc_final(ncls)(s2, degp, b6p.reshape(1, D))
    return out[:, :ncls]


# speedup vs baseline: 4.5863x; 2.0376x over previous
import functools

import jax
import jax.numpy as jnp
from jax import lax
from jax.experimental import pallas as pl
from jax.experimental.pallas import tpu as pltpu
from jax.experimental.pallas import tpu_sc as plsc

N = 10000
D = 128
NCORES = 2
NSUB = 16
NW = NCORES * NSUB
BLK = 128
NBLK = 80
NPAD = N + 112
RPS = NPAD // NSUB
LASTR = N - (NSUB - 1) * RPS
RB = 1000


@functools.partial(
    pl.kernel,
    out_type=jax.ShapeDtypeStruct((NCORES, N, D), jnp.float32),
    mesh=plsc.VectorSubcoreMesh(core_axis_name="c", subcore_axis_name="s"),
    scratch_types=[
        pltpu.VMEM((NBLK * BLK,), jnp.int32),
        pltpu.VMEM((2, BLK), jnp.int32),
        pltpu.VMEM((BLK, D), jnp.float32),
        pltpu.VMEM((BLK, D), jnp.float32),
        pltpu.VMEM_SHARED((NPAD, D), jnp.float32),
        pltpu.SemaphoreType.DMA,
        pltpu.SemaphoreType.DMA,
        pltpu.SemaphoreType.DMA,
        pltpu.SemaphoreType.DMA,
    ],
)
def _agg(y_hbm, src_hbm, dst_hbm, zrows_hbm, out_hbm,
         idx_s, idx_d, rows0, rows1, acc, g0, g1, d0, d1):
    c = lax.axis_index("c")
    s = lax.axis_index("s")
    wid = s * NCORES + c
    base = wid * (NBLK * BLK)

    pltpu.sync_copy(src_hbm.at[pl.ds(base, NBLK * BLK)], idx_s)

    @pl.when(jnp.logical_and(c == 0, s < NSUB - 1))
    def _():
        pltpu.sync_copy(y_hbm.at[pl.ds(s * RPS, RPS)], acc.at[pl.ds(s * RPS, RPS)])

    @pl.when(jnp.logical_and(c == 0, s == NSUB - 1))
    def _():
        pltpu.sync_copy(y_hbm.at[pl.ds(s * RPS, LASTR)], acc.at[pl.ds(s * RPS, LASTR)])

    @pl.when(jnp.logical_and(c != 0, s < NSUB - 1))
    def _():
        pltpu.sync_copy(zrows_hbm, acc.at[pl.ds(s * RPS, RPS)])

    @pl.when(jnp.logical_and(c != 0, s == NSUB - 1))
    def _():
        pltpu.sync_copy(zrows_hbm.at[pl.ds(0, LASTR)], acc.at[pl.ds(s * RPS, LASTR)])

    plsc.subcore_barrier()

    pltpu.async_copy(dst_hbm.at[pl.ds(base, BLK)], idx_d.at[0], d0)
    pltpu.async_copy(dst_hbm.at[pl.ds(base + BLK, BLK)], idx_d.at[1], d1)
    pltpu.async_copy(y_hbm.at[idx_s.at[pl.ds(0, BLK)]], rows0, g0)
    pltpu.async_copy(y_hbm.at[idx_s.at[pl.ds(BLK, BLK)]], rows1, g1)

    def pair(gi, carry):
        i0 = 2 * gi
        i1 = i0 + 1
        pltpu.make_async_copy(y_hbm.at[idx_s.at[pl.ds(0, BLK)]], rows0, g0).wait()
        pltpu.make_async_copy(dst_hbm.at[pl.ds(base, BLK)], idx_d.at[0], d0).wait()
        pltpu.sync_copy(rows0, acc.at[idx_d.at[0]], add=True)

        @pl.when(i0 + 2 < NBLK)
        def _():
            pltpu.async_copy(y_hbm.at[idx_s.at[pl.ds((i0 + 2) * BLK, BLK)]], rows0, g0)
            pltpu.async_copy(dst_hbm.at[pl.ds(base + (i0 + 2) * BLK, BLK)], idx_d.at[0], d0)

        pltpu.make_async_copy(y_hbm.at[idx_s.at[pl.ds(0, BLK)]], rows1, g1).wait()
        pltpu.make_async_copy(dst_hbm.at[pl.ds(base, BLK)], idx_d.at[1], d1).wait()
        pltpu.sync_copy(rows1, acc.at[idx_d.at[1]], add=True)

        @pl.when(i1 + 2 < NBLK)
        def _():
            pltpu.async_copy(y_hbm.at[idx_s.at[pl.ds((i1 + 2) * BLK, BLK)]], rows1, g1)
            pltpu.async_copy(dst_hbm.at[pl.ds(base + (i1 + 2) * BLK, BLK)], idx_d.at[1], d1)

        return carry

    lax.fori_loop(0, NBLK // 2, pair, 0)
    plsc.subcore_barrier()

    @pl.when(s < NSUB - 1)
    def _():
        pltpu.sync_copy(acc.at[pl.ds(s * RPS, RPS)], out_hbm.at[c, pl.ds(s * RPS, RPS)])

    @pl.when(s == NSUB - 1)
    def _():
        pltpu.sync_copy(acc.at[pl.ds(s * RPS, LASTR)], out_hbm.at[c, pl.ds(s * RPS, LASTR)])


@functools.partial(
    pl.kernel,
    out_type=jax.ShapeDtypeStruct((NCORES * NPAD,), jnp.float32),
    mesh=plsc.VectorSubcoreMesh(core_axis_name="c", subcore_axis_name="s"),
    scratch_types=[
        pltpu.VMEM((2, BLK), jnp.int32),
        pltpu.VMEM((1, BLK), jnp.float32),
        pltpu.VMEM((NPAD // NSUB,), jnp.float32),
        pltpu.VMEM_SHARED((NPAD,), jnp.float32),
        pltpu.SemaphoreType.DMA,
        pltpu.SemaphoreType.DMA,
    ],
)
def _deg(dst_hbm, znpad_hbm, ones_hbm, out_hbm, idxb, onesb, bounce, acc1, d0, d1):
    c = lax.axis_index("c")
    s = lax.axis_index("s")
    wid = s * NCORES + c
    base = wid * (NBLK * BLK)
    eps = NPAD // NSUB
    pltpu.sync_copy(ones_hbm, onesb)
    pltpu.sync_copy(znpad_hbm.at[pl.ds(s * eps, eps)], bounce)
    pltpu.sync_copy(bounce, acc1.at[pl.ds(s * eps, eps)])
    plsc.subcore_barrier()

    pltpu.async_copy(dst_hbm.at[pl.ds(base, BLK)], idxb.at[0], d0)
    pltpu.async_copy(dst_hbm.at[pl.ds(base + BLK, BLK)], idxb.at[1], d1)

    def pair(gi, carry):
        i0 = 2 * gi
        i1 = i0 + 1
        pltpu.make_async_copy(dst_hbm.at[pl.ds(base, BLK)], idxb.at[0], d0).wait()
        pltpu.sync_copy(onesb.at[0], acc1.at[idxb.at[0]], add=True)

        @pl.when(i0 + 2 < NBLK)
        def _():
            pltpu.async_copy(dst_hbm.at[pl.ds(base + (i0 + 2) * BLK, BLK)], idxb.at[0], d0)

        pltpu.make_async_copy(dst_hbm.at[pl.ds(base, BLK)], idxb.at[1], d1).wait()
        pltpu.sync_copy(onesb.at[0], acc1.at[idxb.at[1]], add=True)

        @pl.when(i1 + 2 < NBLK)
        def _():
            pltpu.async_copy(dst_hbm.at[pl.ds(base + (i1 + 2) * BLK, BLK)], idxb.at[1], d1)

        return carry

    lax.fori_loop(0, NBLK // 2, pair, 0)
    plsc.subcore_barrier()
    pltpu.sync_copy(acc1.at[pl.ds(s * eps, eps)], bounce)
    pltpu.sync_copy(bounce, out_hbm.at[pl.ds(c * NPAD + s * eps, eps)])


def _dinv_of(degp_ref):
    return lax.rsqrt(degp_ref[0] + degp_ref[1] + 1.0)


def _tc_first_body(x_ref, degp_ref, w_ref, y_ref):
    dinv = _dinv_of(degp_ref)
    y_ref[...] = jnp.dot(x_ref[...], w_ref[...],
                         preferred_element_type=jnp.float32) * dinv


_tc_first = pl.pallas_call(
    _tc_first_body,
    grid=(N // RB,),
    in_specs=[
        pl.BlockSpec((RB, D), lambda i: (i, 0)),
        pl.BlockSpec((NCORES, RB, 1), lambda i: (0, i, 0)),
        pl.BlockSpec((D, D), lambda i: (0, 0)),
    ],
    out_specs=pl.BlockSpec((RB, D), lambda i: (i, 0)),
    out_shape=jax.ShapeDtypeStruct((N, D), jnp.float32),
)


def _tc_mid_body(s2_ref, degp_ref, b_ref, w_ref, y_ref):
    dinv = _dinv_of(degp_ref)
    sagg = s2_ref[0] + s2_ref[1]
    h = jnp.maximum(sagg * dinv + b_ref[0][None, :], 0.0)
    y_ref[...] = jnp.dot(h, w_ref[...],
                         preferred_element_type=jnp.float32) * dinv


_tc_mid = pl.pallas_call(
    _tc_mid_body,
    grid=(N // RB,),
    in_specs=[
        pl.BlockSpec((NCORES, RB, D), lambda i: (0, i, 0)),
        pl.BlockSpec((NCORES, RB, 1), lambda i: (0, i, 0)),
        pl.BlockSpec((1, D), lambda i: (0, 0)),
        pl.BlockSpec((D, D), lambda i: (0, 0)),
    ],
    out_specs=pl.BlockSpec((RB, D), lambda i: (i, 0)),
    out_shape=jax.ShapeDtypeStruct((N, D), jnp.float32),
)


def _make_tc_final(ncls):
    def body(s2_ref, degp_ref, b_ref, out_ref):
        dinv = _dinv_of(degp_ref)
        logits = (s2_ref[0] + s2_ref[1]) * dinv + b_ref[0][None, :]
        col = lax.broadcasted_iota(jnp.int32, (RB, D), 1)
        masked = jnp.where(col < ncls, logits, -1e30)
        m = jnp.max(masked, axis=-1, keepdims=True)
        lse = m + jnp.log(jnp.sum(jnp.exp(masked - m), axis=-1, keepdims=True))
        out_ref[...] = logits - lse

    return pl.pallas_call(
        body,
        grid=(N // RB,),
        in_specs=[
            pl.BlockSpec((NCORES, RB, D), lambda i: (0, i, 0)),
            pl.BlockSpec((NCORES, RB, 1), lambda i: (0, i, 0)),
            pl.BlockSpec((1, D), lambda i: (0, 0)),
        ],
        out_specs=pl.BlockSpec((RB, D), lambda i: (i, 0)),
        out_shape=jax.ShapeDtypeStruct((N, D), jnp.float32),
    )


def kernel(x, edge_index, W1, b1, W2, b2, W3, b3, W4, b4, W5, b5, W6, b6):
    ncls = W6.shape[1]
    w6p = jnp.pad(W6, ((0, 0), (0, D - ncls)))
    b6p = jnp.pad(b6, (0, D - ncls))
    src = edge_index[0]
    dst = edge_index[1]
    e = src.shape[0]
    pad = NW * NBLK * BLK - e
    it = jnp.arange(pad, dtype=src.dtype)
    src_p = jnp.concatenate([src, it % BLK])
    dst_p = jnp.concatenate([dst, N + it % (NPAD - N)])
    zrows = jnp.zeros((RPS, D), jnp.float32)
    znpad = jnp.zeros((NPAD,), jnp.float32)

    degp = _deg(dst_p, znpad, jnp.ones((1, BLK), jnp.float32)).reshape(NCORES, NPAD, 1)
    y = _tc_first(x, degp, W1)
    for bk, wn in ((b1, W2), (b2, W3), (b3, W4), (b4, W5), (b5, w6p)):
        s2 = _agg(y, src_p, dst_p, zrows)
        y = _tc_mid(s2, degp, bk.reshape(1, D), wn)
    s2 = _agg(y, src_p, dst_p, zrows)
    out = _make_tc_final(ncls)(s2, degp, b6p.reshape(1, D))
    return out[:, :ncls]
